# Initial kernel scaffold; baseline (speedup 1.0000x reference)
import jax
import jax.numpy as jnp
from jax.experimental import pallas as pl


def kernel(loc_data, conf_data, loc_data_tch, conf_data_tch, feat_stu, feat_tch, priors, targets):
    raise NotImplementedError("write your pallas kernel here")


# speedup vs baseline: 3.6606x; 3.6606x over previous
import jax
import jax.numpy as jnp
from jax import lax
from jax.experimental import pallas as pl
from jax.experimental.pallas import tpu as pltpu

_VAR0, _VAR1 = 0.1, 0.2
_THRESH = 0.5
_NEGPOS = 3


def _smooth_l1(x):
    ax = jnp.abs(x)
    return jnp.where(ax < 1.0, 0.5 * ax * ax, ax - 0.5)


def _k1_body(T, C, L, P,
             loc_ref, conf_ref, cdt_ref, pri_ref, tgt_ref, fs_ref, ft_ref,
             ll_ref, lc_ref, ld_ref, np_ref, ftm_ref, fsm_ref, coeff_ref):
    b = pl.program_id(0)

    @pl.when(b == 0)
    def _init():
        ll_ref[0, 0] = 0.0
        lc_ref[0, 0] = 0.0
        ld_ref[0, 0] = 0.0
        np_ref[0, 0] = 0.0
        ftm_ref[0, 0] = -jnp.inf
        fsm_ref[0, 0] = -jnp.inf

    pw = pri_ref[...]
    pcx, pcy, pwd, pht = pw[:, 0, :], pw[:, 1, :], pw[:, 2, :], pw[:, 3, :]
    px1 = pcx - pwd * 0.5
    py1 = pcy - pht * 0.5
    px2 = pcx + pwd * 0.5
    py2 = pcy + pht * 0.5
    area_p = (px2 - px1) * (py2 - py1)

    tgt = tgt_ref[0]

    iota_r = lax.broadcasted_iota(jnp.int32, (2, L), 0)
    iota_l = lax.broadcasted_iota(jnp.int32, (2, L), 1)
    iota_p = iota_l * 2 + iota_r

    bto = None
    bti = None
    bpidx = []
    for t in range(T):
        tx1, ty1, tx2, ty2 = tgt[t, 0], tgt[t, 1], tgt[t, 2], tgt[t, 3]
        iw = jnp.maximum(jnp.minimum(tx2, px2) - jnp.maximum(tx1, px1), 0.0)
        ih = jnp.maximum(jnp.minimum(ty2, py2) - jnp.maximum(ty1, py1), 0.0)
        inter = iw * ih
        area_t = (tx2 - tx1) * (ty2 - ty1)
        ov = inter / (area_t + area_p - inter)
        if t == 0:
            bto = ov
            bti = jnp.zeros((2, L), jnp.int32)
        else:
            upd = ov > bto
            bti = jnp.where(upd, t, bti)
            bto = jnp.where(upd, ov, bto)
        m = jnp.max(ov)
        bpidx.append(jnp.min(jnp.where(ov == m, iota_p, P)))
    for t in range(T):
        msk = iota_p == bpidx[t]
        bto = jnp.where(msk, 2.0, bto)
        bti = jnp.where(msk, t, bti)

    mx1 = my1 = mx2 = my2 = lbl = None
    for t in range(T):
        s = bti == t
        if t == 0:
            mx1, my1 = jnp.full((2, L), tgt[0, 0]), jnp.full((2, L), tgt[0, 1])
            mx2, my2 = jnp.full((2, L), tgt[0, 2]), jnp.full((2, L), tgt[0, 3])
            lbl = jnp.full((2, L), tgt[0, 4])
        else:
            mx1 = jnp.where(s, tgt[t, 0], mx1)
            my1 = jnp.where(s, tgt[t, 1], my1)
            mx2 = jnp.where(s, tgt[t, 2], mx2)
            my2 = jnp.where(s, tgt[t, 3], my2)
            lbl = jnp.where(s, tgt[t, 4], lbl)

    conf_t = jnp.where(bto < _THRESH, 0, (lbl + 1.0).astype(jnp.int32))
    pos = conf_t > 0
    posf = pos.astype(jnp.float32)
    npos_i = jnp.sum(conf_t > 0, dtype=jnp.int32)

    g_cx = ((mx1 + mx2) * 0.5 - pcx) / (_VAR0 * pwd)
    g_cy = ((my1 + my2) * 0.5 - pcy) / (_VAR0 * pht)
    g_w = jnp.log((mx2 - mx1) / pwd) / _VAR1
    g_h = jnp.log((my2 - my1) / pht) / _VAR1
    lw = loc_ref[0]
    sl = (_smooth_l1(lw[:, 0, :] - g_cx) + _smooth_l1(lw[:, 1, :] - g_cy)
          + _smooth_l1(lw[:, 2, :] - g_w) + _smooth_l1(lw[:, 3, :] - g_h))
    part_ll = jnp.sum(posf * sl)

    cs = conf_ref[0]
    ms = jnp.max(cs, axis=1, keepdims=True)
    es = jnp.exp(cs - ms)
    ss = jnp.sum(es, axis=1, keepdims=True)
    lss = jnp.log(ss)
    lse = lss[:, 0, :] + ms[:, 0, :]
    log_p = cs - ms - lss
    iota_c = lax.broadcasted_iota(jnp.int32, (2, C, L), 1)
    onehot = iota_c == conf_t[:, None, :]
    gathered = jnp.sum(jnp.where(onehot, cs, 0.0), axis=1)
    ce = lse - gathered
    lcv = jnp.where(pos, 0.0, ce)

    k = jnp.minimum(_NEGPOS * npos_i, P - 1)
    bits = lax.bitcast_convert_type(lcv, jnp.int32)

    def _bis_val(_, lohi):
        lo, hi = lohi
        mid = lo + (hi - lo) // 2
        ok = jnp.sum(bits >= mid, dtype=jnp.int32) >= k
        return jnp.where(ok, mid, lo), jnp.where(ok, hi, mid)

    tau, _ = lax.fori_loop(0, 31, _bis_val,
                           (jnp.int32(0), jnp.int32(0x7F800000)))
    n_gt = jnp.sum(bits > tau, dtype=jnp.int32)
    t_need = k - n_gt
    tie = bits == tau

    def _bis_idx(_, lohi):
        lo, hi = lohi
        mid = lo + (hi - lo) // 2
        ok = jnp.sum(tie & (iota_p <= mid), dtype=jnp.int32) >= t_need
        return jnp.where(ok, lo, mid), jnp.where(ok, mid, hi)

    _, psi = lax.fori_loop(0, 14, _bis_idx,
                           (jnp.int32(-1), jnp.int32(P - 1)))
    neg = (bits > tau) | (tie & (iota_p <= psi) & (t_need > 0))
    selm = pos | neg
    self_ = selm.astype(jnp.float32)
    part_lc = jnp.sum(self_ * ce)

    ct = cdt_ref[0]
    mt = jnp.max(ct, axis=1, keepdims=True)
    et = jnp.exp(ct - mt)
    st = jnp.sum(et, axis=1, keepdims=True)
    log_pt = ct - mt - jnp.log(st)
    p_t = et / st
    kl = p_t * log_pt - p_t * log_p
    kls = jnp.sum(kl, axis=1)
    tqs = jnp.sum(-p_t * log_pt, axis=1)
    w = (1.0 - jnp.exp(-kls - 2.0 * tqs)) ** 2
    part_ld = jnp.sum(self_ * kls * w)

    selw = self_ * w
    coeff_ref[...] = (selw[0:1, :] + selw[1:2, :]).reshape(1, 1, L)
    selc = jnp.maximum(self_[0:1, :], self_[1:2, :])

    cmt = jnp.max(ft_ref[0], axis=0, keepdims=True)
    cms = jnp.max(fs_ref[0], axis=0, keepdims=True)
    pft = jnp.max(jnp.where(selc > 0.0, cmt, -jnp.inf))
    pfs = jnp.max(jnp.where(selc > 0.0, cms, -jnp.inf))

    ll_ref[0, 0] += part_ll
    lc_ref[0, 0] += part_lc
    ld_ref[0, 0] += part_ld
    np_ref[0, 0] += npos_i.astype(jnp.float32)
    ftm_ref[0, 0] = jnp.maximum(ftm_ref[0, 0], pft)
    fsm_ref[0, 0] = jnp.maximum(fsm_ref[0, 0], pfs)


def _k3_body(fs_ref, ft_ref, coeff_ref, fsm_ref, ftm_ref, out_ref):
    b = pl.program_id(0)

    @pl.when(b == 0)
    def _init():
        out_ref[0, 0] = 0.0

    rs = 1.0 / fsm_ref[0, 0]
    rt = 1.0 / ftm_ref[0, 0]
    d = fs_ref[0] * rs - ft_ref[0] * rt
    sl = _smooth_l1(d)
    colsum = jnp.sum(sl, axis=0, keepdims=True)
    out_ref[0, 0] += jnp.sum(coeff_ref[0] * colsum)


def kernel(loc_data, conf_data, loc_data_tch, conf_data_tch,
           feat_stu, feat_tch, priors, targets):
    del loc_data_tch
    B, P, C = conf_data.shape
    Cf, L = feat_stu.shape[1], feat_stu.shape[2]
    T = targets.shape[1]

    locw = loc_data.reshape(B, L, 2, 4).transpose(0, 2, 3, 1)
    confw = conf_data.reshape(B, L, 2, C).transpose(0, 2, 3, 1)
    cdtw = conf_data_tch.reshape(B, L, 2, C).transpose(0, 2, 3, 1)
    priw = priors.reshape(L, 2, 4).transpose(1, 2, 0)

    import functools
    body1 = functools.partial(_k1_body, T, C, L, P)

    sc = jax.ShapeDtypeStruct((1, 1), jnp.float32)
    smem_sc = pl.BlockSpec((1, 1), lambda b: (0, 0), memory_space=pltpu.SMEM)
    ll, lc, ld, npf, ftm, fsm, coeff = pl.pallas_call(
        body1,
        grid=(B,),
        in_specs=[
            pl.BlockSpec((1, 2, 4, L), lambda b: (b, 0, 0, 0)),
            pl.BlockSpec((1, 2, C, L), lambda b: (b, 0, 0, 0)),
            pl.BlockSpec((1, 2, C, L), lambda b: (b, 0, 0, 0)),
            pl.BlockSpec((2, 4, L), lambda b: (0, 0, 0)),
            pl.BlockSpec((1, T, 5), lambda b: (b, 0, 0)),
            pl.BlockSpec((1, Cf, L), lambda b: (b, 0, 0)),
            pl.BlockSpec((1, Cf, L), lambda b: (b, 0, 0)),
        ],
        out_specs=[
            smem_sc, smem_sc, smem_sc, smem_sc, smem_sc, smem_sc,
            pl.BlockSpec((1, 1, L), lambda b: (b, 0, 0)),
        ],
        out_shape=[sc, sc, sc, sc, sc, sc,
                   jax.ShapeDtypeStruct((B, 1, L), jnp.float32)],
        compiler_params=pltpu.CompilerParams(
            dimension_semantics=("arbitrary",)),
    )(locw, confw, cdtw, priw, targets, feat_stu, feat_tch)

    (mim,) = pl.pallas_call(
        _k3_body,
        grid=(B,),
        in_specs=[
            pl.BlockSpec((1, Cf, L), lambda b: (b, 0, 0)),
            pl.BlockSpec((1, Cf, L), lambda b: (b, 0, 0)),
            pl.BlockSpec((1, 1, L), lambda b: (b, 0, 0)),
            smem_sc,
            smem_sc,
        ],
        out_specs=[smem_sc],
        out_shape=[sc],
        compiler_params=pltpu.CompilerParams(
            dimension_semantics=("arbitrary",)),
    )(feat_stu, feat_tch, coeff, fsm, ftm)

    n = jnp.maximum(npf[0, 0], 1.0)
    return (ll[0, 0] / n, lc[0, 0] / n, ld[0, 0] / n, mim[0, 0] / n)
